# Initial kernel scaffold; baseline (speedup 1.0000x reference)
import jax
import jax.numpy as jnp
from jax.experimental import pallas as pl


def kernel(x, edge_index, W1, b1, W2, b2, top_k):
    raise NotImplementedError("write your pallas kernel here")


# speedup vs baseline: 1.3578x; 1.3578x over previous
import functools

import jax
import jax.numpy as jnp
from jax import lax
from jax.experimental import pallas as pl
from jax.experimental.pallas import tpu as pltpu
from jax.experimental.pallas import tpu_sc as plsc

N = 10000
E = 160000
D = 256
HID = 64
TW = 128

_NC = 2
_NS = 16
_NW = _NC * _NS
_L = 16

E_PAD = 163840
_EW = E_PAD // _NW
_CB = 128
_NCH = _EW // _CB

_EPAD_ROWS = E_PAD // 128


def _bf16_round(t):
    u = lax.bitcast_convert_type(t, jnp.int32)
    u = u + 32767 + (lax.shift_right_logical(u, 16) & 1)
    return lax.bitcast_convert_type(u & jnp.int32(-65536), jnp.float32)


def _proj_body(x_ref, w1a_ref, w1b_ref, b1_ref, t_ref):
    xx = x_ref[...]
    p = jnp.dot(xx, w1a_ref[...],
                preferred_element_type=jnp.float32) + b1_ref[...]
    q = jnp.dot(xx, w1b_ref[...],
                preferred_element_type=jnp.float32)
    t_ref[...] = jnp.concatenate([p, q], axis=1)


def _node_projections(x, w1a, w1b, b1row):
    bn = 2000
    return pl.pallas_call(
        _proj_body,
        grid=(N // bn,),
        in_specs=[
            pl.BlockSpec((bn, D), lambda i: (i, 0)),
            pl.BlockSpec((D, HID), lambda i: (0, 0)),
            pl.BlockSpec((D, HID), lambda i: (0, 0)),
            pl.BlockSpec((1, HID), lambda i: (0, 0)),
        ],
        out_specs=pl.BlockSpec((bn, TW), lambda i: (i, 0)),
        out_shape=jax.ShapeDtypeStruct((N, TW), jnp.float32),
    )(x, w1a, w1b, b1row)


def _edge_scores_sc(t_tab, src_pad, dst_pad, w2flat):
    mesh = plsc.VectorSubcoreMesh(core_axis_name="c", subcore_axis_name="s")

    @functools.partial(
        pl.kernel,
        out_type=jax.ShapeDtypeStruct((E_PAD,), jnp.float32),
        mesh=mesh,
        compiler_params=pltpu.CompilerParams(needs_layout_passes=False),
        scratch_types=[
            pltpu.VMEM((_CB,), jnp.int32),
            pltpu.VMEM((_CB,), jnp.int32),
            pltpu.VMEM((_CB, TW), jnp.float32),
            pltpu.VMEM((_CB, TW), jnp.float32),
            pltpu.VMEM((_CB,), jnp.float32),
            pltpu.VMEM((HID,), jnp.float32),
            pltpu.SemaphoreType.DMA,
            pltpu.SemaphoreType.DMA,
        ],
    )
    def k(t_hbm, src_hbm, dst_hbm, w2_hbm, out_hbm,
          sidx, didx, srows, drows, vout, w2v, sem_s, sem_d):
        wid = lax.axis_index("s") * _NC + lax.axis_index("c")
        pltpu.sync_copy(w2_hbm, w2v)
        for i in range(HID // _L):
            w2v[pl.ds(i * _L, _L)] = _bf16_round(w2v[pl.ds(i * _L, _L)])

        def chunk_body(c, carry):
            base = wid * _EW + c * _CB
            pltpu.sync_copy(src_hbm.at[pl.ds(base, _CB)], sidx)
            pltpu.sync_copy(dst_hbm.at[pl.ds(base, _CB)], didx)
            cp_s = pltpu.async_copy(t_hbm.at[sidx], srows, sem_s)
            cp_d = pltpu.async_copy(t_hbm.at[didx], drows, sem_d)
            cp_s.wait()
            cp_d.wait()

            def group_body(g, carry2):
                rows = g * _L + lax.iota(jnp.int32, _L)

                def d_body(d, acc):
                    cols = jnp.full((_L,), d, jnp.int32)
                    pd = plsc.load_gather(srows, [rows, cols])
                    qd = plsc.load_gather(drows, [rows, cols + HID])
                    w2d = plsc.load_gather(w2v, [cols])
                    h = jnp.maximum(pd + qd, 0.0)
                    return acc + _bf16_round(h) * w2d

                acc = lax.fori_loop(0, HID, d_body,
                                    jnp.zeros((_L,), jnp.float32))
                vout[pl.ds(g * _L, _L)] = acc
                return carry2

            lax.fori_loop(0, _CB // _L, group_body, 0)
            pltpu.sync_copy(vout, out_hbm.at[pl.ds(base, _CB)])
            return carry

        lax.fori_loop(0, _NCH, chunk_body, 0)

    return k(t_tab, src_pad, dst_pad, w2flat)


def _mask_body(v_ref, k_ref, b2_ref, em_ref, hard_ref, sp_ref):
    v = v_ref[...] + b2_ref[0, 0]
    em = jax.nn.sigmoid(v)
    bits = lax.bitcast_convert_type(em, jnp.int32)
    rows = lax.broadcasted_iota(jnp.int32, v.shape, 0)
    cols = lax.broadcasted_iota(jnp.int32, v.shape, 1)
    flat = rows * v.shape[1] + cols
    key = jnp.where(flat < E, bits, -1)
    kth1 = k_ref[0, 0]

    def bit_body(i, t):
        cand = t | jnp.left_shift(jnp.int32(1), 30 - i)
        cnt = jnp.sum((key >= cand).astype(jnp.int32))
        return jnp.where(cnt >= kth1, cand, t)

    t_bits = lax.fori_loop(0, 31, bit_body, jnp.int32(0))
    hard = (key > t_bits).astype(jnp.float32)
    em_ref[...] = em
    hard_ref[...] = hard
    sp_ref[...] = em * hard


def _mask_outputs(values_pad2d, kth1_arr, b2arr):
    return pl.pallas_call(
        _mask_body,
        in_specs=[
            pl.BlockSpec(memory_space=pltpu.MemorySpace.VMEM),
            pl.BlockSpec(memory_space=pltpu.MemorySpace.SMEM),
            pl.BlockSpec(memory_space=pltpu.MemorySpace.SMEM),
        ],
        out_specs=[
            pl.BlockSpec(memory_space=pltpu.MemorySpace.VMEM),
            pl.BlockSpec(memory_space=pltpu.MemorySpace.VMEM),
            pl.BlockSpec(memory_space=pltpu.MemorySpace.VMEM),
        ],
        out_shape=[
            jax.ShapeDtypeStruct((_EPAD_ROWS, 128), jnp.float32),
            jax.ShapeDtypeStruct((_EPAD_ROWS, 128), jnp.float32),
            jax.ShapeDtypeStruct((_EPAD_ROWS, 128), jnp.float32),
        ],
    )(values_pad2d, kth1_arr, b2arr)


def kernel(x, edge_index, W1, b1, W2, b2, top_k):
    xb = x.astype(jnp.bfloat16)
    w1a = W1[:D].astype(jnp.bfloat16)
    w1b = W1[D:].astype(jnp.bfloat16)
    b1row = b1.reshape(1, HID)
    w2flat = W2.reshape(HID)

    t_tab = _node_projections(xb, w1a, w1b, b1row)

    src = edge_index[0]
    dst = edge_index[1]
    pad = E_PAD - E
    src_pad = jnp.concatenate([src, jnp.zeros((pad,), jnp.int32)])
    dst_pad = jnp.concatenate([dst, jnp.zeros((pad,), jnp.int32)])

    values = _edge_scores_sc(t_tab, src_pad, dst_pad, w2flat)

    kth1 = (jnp.minimum(jnp.asarray(top_k, jnp.int32), E - 1) + 1).reshape(1, 1)
    b2arr = b2.astype(jnp.float32).reshape(1, 1)
    em2d, hard2d, sp2d = _mask_outputs(values.reshape(_EPAD_ROWS, 128),
                                       kth1, b2arr)

    em = em2d.reshape(-1)[:E]
    hard = hard2d.reshape(-1)[:E]
    sp = sp2d.reshape(-1)[:E]
    return (em, hard, sp)


# speedup vs baseline: 2.1652x; 1.5946x over previous
import functools

import jax
import jax.numpy as jnp
from jax import lax
from jax.experimental import pallas as pl
from jax.experimental.pallas import tpu as pltpu
from jax.experimental.pallas import tpu_sc as plsc

N = 10000
E = 160000
D = 256
HID = 64
TW = 128

_NC = 2
_NS = 16
_NW = _NC * _NS
_L = 16

E_PAD = 163840
_EW = E_PAD // _NW
_CB = 128
_NCH = _EW // _CB

_EPAD_ROWS = E_PAD // 128


def _bf16_round(t):
    u = lax.bitcast_convert_type(t, jnp.int32)
    u = u + 32767 + (lax.shift_right_logical(u, 16) & 1)
    return lax.bitcast_convert_type(u & jnp.int32(-65536), jnp.float32)


def _proj_body(x_ref, w1a_ref, w1b_ref, b1_ref, t_ref):
    xx = x_ref[...]
    p = jnp.dot(xx, w1a_ref[...],
                preferred_element_type=jnp.float32) + b1_ref[...]
    q = jnp.dot(xx, w1b_ref[...],
                preferred_element_type=jnp.float32)
    t_ref[...] = jnp.concatenate([p, q], axis=1)


def _node_projections(x, w1a, w1b, b1row):
    bn = 2000
    return pl.pallas_call(
        _proj_body,
        grid=(N // bn,),
        in_specs=[
            pl.BlockSpec((bn, D), lambda i: (i, 0)),
            pl.BlockSpec((D, HID), lambda i: (0, 0)),
            pl.BlockSpec((D, HID), lambda i: (0, 0)),
            pl.BlockSpec((1, HID), lambda i: (0, 0)),
        ],
        out_specs=pl.BlockSpec((bn, TW), lambda i: (i, 0)),
        out_shape=jax.ShapeDtypeStruct((N, TW), jnp.float32),
    )(x, w1a, w1b, b1row)


def _edge_scores_sc(t_tab, src_pad, dst_pad, w2flat):
    mesh = plsc.VectorSubcoreMesh(core_axis_name="c", subcore_axis_name="s")
    npair = _NCH // 2

    @functools.partial(
        pl.kernel,
        out_type=jax.ShapeDtypeStruct((E_PAD,), jnp.float32),
        mesh=mesh,
        compiler_params=pltpu.CompilerParams(needs_layout_passes=False),
        scratch_types=[
            pltpu.VMEM((_CB,), jnp.int32),
            pltpu.VMEM((_CB,), jnp.int32),
            pltpu.VMEM((_CB,), jnp.int32),
            pltpu.VMEM((_CB,), jnp.int32),
            pltpu.VMEM((_CB, TW), jnp.float32),
            pltpu.VMEM((_CB, TW), jnp.float32),
            pltpu.VMEM((_CB, TW), jnp.float32),
            pltpu.VMEM((_CB, TW), jnp.float32),
            pltpu.VMEM((_EW,), jnp.float32),
            pltpu.VMEM((HID,), jnp.float32),
            pltpu.SemaphoreType.DMA,
            pltpu.SemaphoreType.DMA,
            pltpu.SemaphoreType.DMA,
            pltpu.SemaphoreType.DMA,
        ],
    )
    def k(t_hbm, src_hbm, dst_hbm, w2_hbm, out_hbm,
          sidx_a, didx_a, sidx_b, didx_b, sr_a, dr_a, sr_b, dr_b, vout, w2v,
          sem_sa, sem_da, sem_sb, sem_db):
        wid = lax.axis_index("s") * _NC + lax.axis_index("c")
        pltpu.sync_copy(w2_hbm, w2v)
        for i in range(HID // _L):
            w2v[pl.ds(i * _L, _L)] = _bf16_round(w2v[pl.ds(i * _L, _L)])
        iota16 = lax.iota(jnp.int32, _L)

        def start_gather(c, sidx, didx, sr, dr, sem_s, sem_d):
            base = wid * _EW + c * _CB
            pltpu.sync_copy(src_hbm.at[pl.ds(base, _CB)], sidx)
            pltpu.sync_copy(dst_hbm.at[pl.ds(base, _CB)], didx)
            pltpu.async_copy(t_hbm.at[sidx], sr, sem_s)
            pltpu.async_copy(t_hbm.at[didx], dr, sem_d)

        def wait_gather(sidx, didx, sr, dr, sem_s, sem_d):
            pltpu.make_async_copy(t_hbm.at[sidx], sr, sem_s).wait()
            pltpu.make_async_copy(t_hbm.at[didx], dr, sem_d).wait()

        def compute(c, sr, dr):
            def g_body(g, carry):
                rows = g * _L + iota16

                def d_body(d, acc):
                    cols = jnp.full((_L,), d, jnp.int32)
                    pd = plsc.load_gather(sr, [rows, cols])
                    qd = plsc.load_gather(dr, [rows, cols + HID])
                    w2d = plsc.load_gather(w2v, [cols])
                    h = jnp.maximum(pd + qd, 0.0)
                    return acc + _bf16_round(h) * w2d

                acc = lax.fori_loop(0, HID, d_body,
                                    jnp.zeros((_L,), jnp.float32))
                vout[pl.ds(c * _CB + g * _L, _L)] = acc
                return carry

            lax.fori_loop(0, _CB // _L, g_body, 0)

        start_gather(0, sidx_a, didx_a, sr_a, dr_a, sem_sa, sem_da)

        def pair_body(i, carry):
            c_a = 2 * i
            c_b = 2 * i + 1
            start_gather(c_b, sidx_b, didx_b, sr_b, dr_b, sem_sb, sem_db)
            wait_gather(sidx_a, didx_a, sr_a, dr_a, sem_sa, sem_da)
            compute(c_a, sr_a, dr_a)

            @pl.when(i < npair - 1)
            def _prefetch():
                start_gather(c_a + 2, sidx_a, didx_a, sr_a, dr_a,
                             sem_sa, sem_da)

            wait_gather(sidx_b, didx_b, sr_b, dr_b, sem_sb, sem_db)
            compute(c_b, sr_b, dr_b)
            return carry

        lax.fori_loop(0, npair, pair_body, 0)
        pltpu.sync_copy(vout, out_hbm.at[pl.ds(wid * _EW, _EW)])

    return k(t_tab, src_pad, dst_pad, w2flat)


def _mask_body(v_ref, k_ref, b2_ref, em_ref, hard_ref, sp_ref):
    v = v_ref[...] + b2_ref[0, 0]
    em = jax.nn.sigmoid(v)
    bits = lax.bitcast_convert_type(em, jnp.int32)
    rows = lax.broadcasted_iota(jnp.int32, v.shape, 0)
    cols = lax.broadcasted_iota(jnp.int32, v.shape, 1)
    flat = rows * v.shape[1] + cols
    key = jnp.where(flat < E, bits, -1)
    kth1 = k_ref[0, 0]

    def bit_body(i, t):
        cand = t | jnp.left_shift(jnp.int32(1), 30 - i)
        cnt = jnp.sum((key >= cand).astype(jnp.int32))
        return jnp.where(cnt >= kth1, cand, t)

    t_bits = lax.fori_loop(0, 31, bit_body, jnp.int32(0))
    hard = (key > t_bits).astype(jnp.float32)
    em_ref[...] = em
    hard_ref[...] = hard
    sp_ref[...] = em * hard


def _mask_outputs(values_pad2d, kth1_arr, b2arr):
    return pl.pallas_call(
        _mask_body,
        in_specs=[
            pl.BlockSpec(memory_space=pltpu.MemorySpace.VMEM),
            pl.BlockSpec(memory_space=pltpu.MemorySpace.SMEM),
            pl.BlockSpec(memory_space=pltpu.MemorySpace.SMEM),
        ],
        out_specs=[
            pl.BlockSpec(memory_space=pltpu.MemorySpace.VMEM),
            pl.BlockSpec(memory_space=pltpu.MemorySpace.VMEM),
            pl.BlockSpec(memory_space=pltpu.MemorySpace.VMEM),
        ],
        out_shape=[
            jax.ShapeDtypeStruct((_EPAD_ROWS, 128), jnp.float32),
            jax.ShapeDtypeStruct((_EPAD_ROWS, 128), jnp.float32),
            jax.ShapeDtypeStruct((_EPAD_ROWS, 128), jnp.float32),
        ],
    )(values_pad2d, kth1_arr, b2arr)


def kernel(x, edge_index, W1, b1, W2, b2, top_k):
    xb = x.astype(jnp.bfloat16)
    w1a = W1[:D].astype(jnp.bfloat16)
    w1b = W1[D:].astype(jnp.bfloat16)
    b1row = b1.reshape(1, HID)
    w2flat = W2.reshape(HID)

    t_tab = _node_projections(xb, w1a, w1b, b1row)

    src = edge_index[0]
    dst = edge_index[1]
    pad = E_PAD - E
    src_pad = jnp.concatenate([src, jnp.zeros((pad,), jnp.int32)])
    dst_pad = jnp.concatenate([dst, jnp.zeros((pad,), jnp.int32)])

    values = _edge_scores_sc(t_tab, src_pad, dst_pad, w2flat)

    kth1 = (jnp.minimum(jnp.asarray(top_k, jnp.int32), E - 1) + 1).reshape(1, 1)
    b2arr = b2.astype(jnp.float32).reshape(1, 1)
    em2d, hard2d, sp2d = _mask_outputs(values.reshape(_EPAD_ROWS, 128),
                                       kth1, b2arr)

    em = em2d.reshape(-1)[:E]
    hard = hard2d.reshape(-1)[:E]
    sp = sp2d.reshape(-1)[:E]
    return (em, hard, sp)


# speedup vs baseline: 2.2077x; 1.0196x over previous
import functools

import jax
import jax.numpy as jnp
from jax import lax
from jax.experimental import pallas as pl
from jax.experimental.pallas import tpu as pltpu
from jax.experimental.pallas import tpu_sc as plsc

N = 10000
E = 160000
D = 256
HID = 64
TW = 128

_NC = 2
_NS = 16
_NW = _NC * _NS
_L = 16

E_PAD = 163840
_EW = E_PAD // _NW
_CB = 128
_NCH = _EW // _CB

_EPAD_ROWS = E_PAD // 128


def _bf16_round(t):
    u = lax.bitcast_convert_type(t, jnp.int32)
    u = u + 32767 + (lax.shift_right_logical(u, 16) & 1)
    return lax.bitcast_convert_type(u & jnp.int32(-65536), jnp.float32)


def _proj_body(x_ref, w1a_ref, w1b_ref, b1_ref, t_ref):
    xx = x_ref[...]
    p = jnp.dot(xx, w1a_ref[...],
                preferred_element_type=jnp.float32) + b1_ref[...]
    q = jnp.dot(xx, w1b_ref[...],
                preferred_element_type=jnp.float32)
    t_ref[...] = jnp.concatenate([p, q], axis=1)


def _node_projections(x, w1a, w1b, b1row):
    bn = 2000
    return pl.pallas_call(
        _proj_body,
        grid=(N // bn,),
        in_specs=[
            pl.BlockSpec((bn, D), lambda i: (i, 0)),
            pl.BlockSpec((D, HID), lambda i: (0, 0)),
            pl.BlockSpec((D, HID), lambda i: (0, 0)),
            pl.BlockSpec((1, HID), lambda i: (0, 0)),
        ],
        out_specs=pl.BlockSpec((bn, TW), lambda i: (i, 0)),
        out_shape=jax.ShapeDtypeStruct((N, TW), jnp.float32),
    )(x, w1a, w1b, b1row)


def _edge_scores_sc(t_tab, src_pad, dst_pad, w2flat):
    mesh = plsc.VectorSubcoreMesh(core_axis_name="c", subcore_axis_name="s")
    npair = _NCH // 2

    @functools.partial(
        pl.kernel,
        out_type=jax.ShapeDtypeStruct((E_PAD,), jnp.float32),
        mesh=mesh,
        compiler_params=pltpu.CompilerParams(needs_layout_passes=False),
        scratch_types=[
            pltpu.VMEM((_CB,), jnp.int32),
            pltpu.VMEM((_CB,), jnp.int32),
            pltpu.VMEM((_CB,), jnp.int32),
            pltpu.VMEM((_CB,), jnp.int32),
            pltpu.VMEM((_CB, TW), jnp.float32),
            pltpu.VMEM((_CB, TW), jnp.float32),
            pltpu.VMEM((_CB, TW), jnp.float32),
            pltpu.VMEM((_CB, TW), jnp.float32),
            pltpu.VMEM((_EW,), jnp.float32),
            pltpu.VMEM((HID,), jnp.float32),
            pltpu.SemaphoreType.DMA,
            pltpu.SemaphoreType.DMA,
            pltpu.SemaphoreType.DMA,
            pltpu.SemaphoreType.DMA,
        ],
    )
    def k(t_hbm, src_hbm, dst_hbm, w2_hbm, out_hbm,
          sidx_a, didx_a, sidx_b, didx_b, sr_a, dr_a, sr_b, dr_b, vout, w2v,
          sem_sa, sem_da, sem_sb, sem_db):
        wid = lax.axis_index("s") * _NC + lax.axis_index("c")
        pltpu.sync_copy(w2_hbm, w2v)
        for i in range(HID // _L):
            w2v[pl.ds(i * _L, _L)] = _bf16_round(w2v[pl.ds(i * _L, _L)])
        iota16 = lax.iota(jnp.int32, _L)

        def start_gather(c, sidx, didx, sr, dr, sem_s, sem_d):
            base = wid * _EW + c * _CB
            pltpu.sync_copy(src_hbm.at[pl.ds(base, _CB)], sidx)
            pltpu.sync_copy(dst_hbm.at[pl.ds(base, _CB)], didx)
            pltpu.async_copy(t_hbm.at[sidx], sr, sem_s)
            pltpu.async_copy(t_hbm.at[didx], dr, sem_d)

        def wait_gather(sidx, didx, sr, dr, sem_s, sem_d):
            pltpu.make_async_copy(t_hbm.at[sidx], sr, sem_s).wait()
            pltpu.make_async_copy(t_hbm.at[didx], dr, sem_d).wait()

        def compute(c, sr, dr):
            def g_body(g, carry):
                rows = g * _L + iota16

                def d_body(d, acc):
                    cols = jnp.full((_L,), d, jnp.int32)
                    pd = plsc.load_gather(sr, [rows, cols])
                    qd = plsc.load_gather(dr, [rows, cols + HID])
                    w2d = plsc.load_gather(w2v, [cols])
                    h = jnp.maximum(pd + qd, 0.0)
                    return acc + _bf16_round(h) * w2d

                acc = lax.fori_loop(0, HID, d_body,
                                    jnp.zeros((_L,), jnp.float32),
                                    unroll=8)
                vout[pl.ds(c * _CB + g * _L, _L)] = acc
                return carry

            lax.fori_loop(0, _CB // _L, g_body, 0)

        start_gather(0, sidx_a, didx_a, sr_a, dr_a, sem_sa, sem_da)

        def pair_body(i, carry):
            c_a = 2 * i
            c_b = 2 * i + 1
            start_gather(c_b, sidx_b, didx_b, sr_b, dr_b, sem_sb, sem_db)
            wait_gather(sidx_a, didx_a, sr_a, dr_a, sem_sa, sem_da)
            compute(c_a, sr_a, dr_a)

            @pl.when(i < npair - 1)
            def _prefetch():
                start_gather(c_a + 2, sidx_a, didx_a, sr_a, dr_a,
                             sem_sa, sem_da)

            wait_gather(sidx_b, didx_b, sr_b, dr_b, sem_sb, sem_db)
            compute(c_b, sr_b, dr_b)
            return carry

        lax.fori_loop(0, npair, pair_body, 0)
        pltpu.sync_copy(vout, out_hbm.at[pl.ds(wid * _EW, _EW)])

    return k(t_tab, src_pad, dst_pad, w2flat)


def _mask_body(v_ref, k_ref, b2_ref, em_ref, hard_ref, sp_ref):
    v = v_ref[...] + b2_ref[0, 0]
    em = jax.nn.sigmoid(v)
    bits = lax.bitcast_convert_type(em, jnp.int32)
    rows = lax.broadcasted_iota(jnp.int32, v.shape, 0)
    cols = lax.broadcasted_iota(jnp.int32, v.shape, 1)
    flat = rows * v.shape[1] + cols
    key = jnp.where(flat < E, bits, -1)
    kth1 = k_ref[0, 0]

    def bit_body(i, t):
        cand = t | jnp.left_shift(jnp.int32(1), 30 - i)
        cnt = jnp.sum((key >= cand).astype(jnp.int32))
        return jnp.where(cnt >= kth1, cand, t)

    t_bits = lax.fori_loop(0, 31, bit_body, jnp.int32(0))
    hard = (key > t_bits).astype(jnp.float32)
    em_ref[...] = em
    hard_ref[...] = hard
    sp_ref[...] = em * hard


def _mask_outputs(values_pad2d, kth1_arr, b2arr):
    return pl.pallas_call(
        _mask_body,
        in_specs=[
            pl.BlockSpec(memory_space=pltpu.MemorySpace.VMEM),
            pl.BlockSpec(memory_space=pltpu.MemorySpace.SMEM),
            pl.BlockSpec(memory_space=pltpu.MemorySpace.SMEM),
        ],
        out_specs=[
            pl.BlockSpec(memory_space=pltpu.MemorySpace.VMEM),
            pl.BlockSpec(memory_space=pltpu.MemorySpace.VMEM),
            pl.BlockSpec(memory_space=pltpu.MemorySpace.VMEM),
        ],
        out_shape=[
            jax.ShapeDtypeStruct((_EPAD_ROWS, 128), jnp.float32),
            jax.ShapeDtypeStruct((_EPAD_ROWS, 128), jnp.float32),
            jax.ShapeDtypeStruct((_EPAD_ROWS, 128), jnp.float32),
        ],
    )(values_pad2d, kth1_arr, b2arr)


def kernel(x, edge_index, W1, b1, W2, b2, top_k):
    xb = x.astype(jnp.bfloat16)
    w1a = W1[:D].astype(jnp.bfloat16)
    w1b = W1[D:].astype(jnp.bfloat16)
    b1row = b1.reshape(1, HID)
    w2flat = W2.reshape(HID)

    t_tab = _node_projections(xb, w1a, w1b, b1row)

    src = edge_index[0]
    dst = edge_index[1]
    pad = E_PAD - E
    src_pad = jnp.concatenate([src, jnp.zeros((pad,), jnp.int32)])
    dst_pad = jnp.concatenate([dst, jnp.zeros((pad,), jnp.int32)])

    values = _edge_scores_sc(t_tab, src_pad, dst_pad, w2flat)

    kth1 = (jnp.minimum(jnp.asarray(top_k, jnp.int32), E - 1) + 1).reshape(1, 1)
    b2arr = b2.astype(jnp.float32).reshape(1, 1)
    em2d, hard2d, sp2d = _mask_outputs(values.reshape(_EPAD_ROWS, 128),
                                       kth1, b2arr)

    em = em2d.reshape(-1)[:E]
    hard = hard2d.reshape(-1)[:E]
    sp = sp2d.reshape(-1)[:E]
    return (em, hard, sp)
